# Initial kernel scaffold; baseline (speedup 1.0000x reference)
import functools

import jax
import jax.numpy as jnp
import numpy as np
from jax import lax
from jax.experimental import pallas as pl
from jax.experimental.pallas import tpu as pltpu
from jax.experimental.pallas import tpu_sc as plsc

VOCAB = 1000000
EMBED = 64
SEQ_LEN = 200
N_SEG = 3

NC = 2
NS = 16
NW = NC * NS

B_TOTAL = 4096 * SEQ_LEN
ROWS_PER_W = B_TOTAL // NW
CHUNK = 128
NCHUNKS = ROWS_PER_W // CHUNK
LANES = 16
CSL = EMBED // LANES


def _make_pe(max_len, d):
    position = jnp.arange(max_len, dtype=jnp.float32)[:, None]
    div_term = jnp.exp(
        jnp.arange(0, d, 2, dtype=jnp.float32) * (-np.log(10000.0) / d)
    )
    pe = jnp.zeros((max_len, d), dtype=jnp.float32)
    pe = pe.at[:, 0::2].set(jnp.sin(position * div_term))
    pe = pe.at[:, 1::2].set(jnp.cos(position * div_term))
    return pe


@functools.partial(
    pl.kernel,
    out_type=jax.ShapeDtypeStruct((B_TOTAL, EMBED), jnp.float32),
    mesh=plsc.VectorSubcoreMesh(core_axis_name="c", subcore_axis_name="s"),
    scratch_types=[
        pltpu.VMEM((CHUNK,), jnp.int32),
        pltpu.VMEM((CHUNK,), jnp.int32),
        pltpu.VMEM((CHUNK,), jnp.int32),
        pltpu.VMEM((CHUNK, EMBED), jnp.float32),
        pltpu.VMEM((CHUNK, EMBED), jnp.float32),
        pltpu.SemaphoreType.DMA,
        pltpu.SemaphoreType.DMA,
    ],
)
def _sc_embed(seq_hbm, seg_hbm, tok_hbm, comb_hbm, out_hbm,
              sidx_v, seg_v, cidx_v, tok_v, cmb_v, sem_t, sem_c):
    wid = lax.axis_index("s") * NC + lax.axis_index("c")
    wbase = wid * ROWS_PER_W

    def chunk_body(c, carry):
        base = wbase + c * CHUNK
        pltpu.sync_copy(seq_hbm.at[pl.ds(base, CHUNK)], sidx_v)
        pltpu.sync_copy(seg_hbm.at[pl.ds(base, CHUNK)], seg_v)

        start = lax.rem(c * CHUNK, SEQ_LEN)
        iota = lax.iota(jnp.int32, LANES)

        def idx_body(k, _):
            off = k * LANES
            traw = start + off + iota
            t = jnp.where(traw >= SEQ_LEN, traw - SEQ_LEN, traw)
            cidx_v[pl.ds(off, LANES)] = seg_v[pl.ds(off, LANES)] * SEQ_LEN + t
            return 0

        lax.fori_loop(0, CHUNK // LANES, idx_body, 0)

        cp_t = pltpu.async_copy(tok_hbm.at[sidx_v], tok_v, sem_t)
        cp_c = pltpu.async_copy(comb_hbm.at[cidx_v], cmb_v, sem_c)
        cp_t.wait()
        cp_c.wait()

        def add_body(r, _):
            for cc in range(CSL):
                sl = pl.ds(cc * LANES, LANES)
                tok_v[r, sl] = tok_v[r, sl] + cmb_v[r, sl]
            return 0

        lax.fori_loop(0, CHUNK, add_body, 0)

        pltpu.sync_copy(tok_v, out_hbm.at[pl.ds(base, CHUNK)])
        return carry

    lax.fori_loop(0, NCHUNKS, chunk_body, 0)


def kernel(sequence, segment_label, token_table, segment_table):
    nb, sl = sequence.shape
    pe = _make_pe(512, EMBED)[:sl]
    comb = (segment_table[:, None, :] + pe[None, :, :]).reshape(N_SEG * sl, EMBED)
    seq_flat = sequence.reshape(-1).astype(jnp.int32)
    seg_flat = segment_label.reshape(-1).astype(jnp.int32)
    out = _sc_embed(seq_flat, seg_flat, token_table, comb)
    return out.reshape(nb, sl, EMBED)


# speedup vs baseline: 2.0372x; 2.0372x over previous
import functools

import jax
import jax.numpy as jnp
import numpy as np
from jax import lax
from jax.experimental import pallas as pl
from jax.experimental.pallas import tpu as pltpu
from jax.experimental.pallas import tpu_sc as plsc

VOCAB = 1000000
EMBED = 64
SEQ_LEN = 200
N_SEG = 3

NC = 2
NS = 16
NW = NC * NS

B_TOTAL = 4096 * SEQ_LEN
ROWS_PER_W = B_TOTAL // NW
CHUNK = 128
NCHUNKS = ROWS_PER_W // CHUNK
LANES = 16
CSL = EMBED // LANES


def _make_pe(max_len, d):
    position = jnp.arange(max_len, dtype=jnp.float32)[:, None]
    div_term = jnp.exp(
        jnp.arange(0, d, 2, dtype=jnp.float32) * (-np.log(10000.0) / d)
    )
    pe = jnp.zeros((max_len, d), dtype=jnp.float32)
    pe = pe.at[:, 0::2].set(jnp.sin(position * div_term))
    pe = pe.at[:, 1::2].set(jnp.cos(position * div_term))
    return pe


@functools.partial(
    pl.kernel,
    out_type=jax.ShapeDtypeStruct((B_TOTAL, EMBED), jnp.float32),
    mesh=plsc.VectorSubcoreMesh(core_axis_name="c", subcore_axis_name="s"),
    scratch_types=[
        pltpu.VMEM((CHUNK,), jnp.int32),
        pltpu.VMEM((CHUNK,), jnp.int32),
        pltpu.VMEM((CHUNK,), jnp.int32),
        pltpu.VMEM((CHUNK, EMBED), jnp.float32),
        pltpu.VMEM((CHUNK, EMBED), jnp.float32),
        pltpu.SemaphoreType.DMA,
        pltpu.SemaphoreType.DMA,
    ],
    compiler_params=pltpu.CompilerParams(use_tc_tiling_on_sc=False),
)
def _sc_embed(seq_hbm, seg_hbm, tok_hbm, comb_hbm, out_hbm,
              sidx_v, seg_v, cidx_v, tok_v, cmb_v, sem_t, sem_c):
    wid = lax.axis_index("s") * NC + lax.axis_index("c")
    wbase = wid * ROWS_PER_W

    def chunk_body(c, carry):
        base = wbase + c * CHUNK
        pltpu.sync_copy(seq_hbm.at[pl.ds(base, CHUNK)], sidx_v)
        pltpu.sync_copy(seg_hbm.at[pl.ds(base, CHUNK)], seg_v)

        start = lax.rem(c * CHUNK, SEQ_LEN)
        iota = lax.iota(jnp.int32, LANES)

        def idx_body(k, _):
            off = k * LANES
            traw = start + off + iota
            t = jnp.where(traw >= SEQ_LEN, traw - SEQ_LEN, traw)
            cidx_v[pl.ds(off, LANES)] = seg_v[pl.ds(off, LANES)] * SEQ_LEN + t
            return 0

        lax.fori_loop(0, CHUNK // LANES, idx_body, 0)

        cp_t = pltpu.async_copy(tok_hbm.at[sidx_v], tok_v, sem_t)
        cp_c = pltpu.async_copy(comb_hbm.at[cidx_v], cmb_v, sem_c)
        cp_t.wait()
        cp_c.wait()

        def add_body(r, _):
            for cc in range(CSL):
                sl = pl.ds(cc * LANES, LANES)
                tok_v[r, sl] = tok_v[r, sl] + cmb_v[r, sl]
            return 0

        lax.fori_loop(0, CHUNK, add_body, 0)

        pltpu.sync_copy(tok_v, out_hbm.at[pl.ds(base, CHUNK)])
        return carry

    lax.fori_loop(0, NCHUNKS, chunk_body, 0)


def kernel(sequence, segment_label, token_table, segment_table):
    nb, sl = sequence.shape
    pe = _make_pe(512, EMBED)[:sl]
    comb = (segment_table[:, None, :] + pe[None, :, :]).reshape(N_SEG * sl, EMBED)
    seq_flat = sequence.reshape(-1).astype(jnp.int32)
    seg_flat = segment_label.reshape(-1).astype(jnp.int32)
    out = _sc_embed(seq_flat, seg_flat, token_table, comb)
    return out.reshape(nb, sl, EMBED)


# speedup vs baseline: 2.3928x; 1.1745x over previous
import functools

import jax
import jax.numpy as jnp
import numpy as np
from jax import lax
from jax.experimental import pallas as pl
from jax.experimental.pallas import tpu as pltpu
from jax.experimental.pallas import tpu_sc as plsc

VOCAB = 1000000
EMBED = 64
SEQ_LEN = 200
N_SEG = 3

NC = 2
NS = 16
NW = NC * NS

B_TOTAL = 4096 * SEQ_LEN
ROWS_PER_W = B_TOTAL // NW
CHUNK = 128
NCHUNKS = ROWS_PER_W // CHUNK
LANES = 16
CSL = EMBED // LANES
ROW_UNROLL = 4


def _make_pe(max_len, d):
    position = jnp.arange(max_len, dtype=jnp.float32)[:, None]
    div_term = jnp.exp(
        jnp.arange(0, d, 2, dtype=jnp.float32) * (-np.log(10000.0) / d)
    )
    pe = jnp.zeros((max_len, d), dtype=jnp.float32)
    pe = pe.at[:, 0::2].set(jnp.sin(position * div_term))
    pe = pe.at[:, 1::2].set(jnp.cos(position * div_term))
    return pe


@functools.partial(
    pl.kernel,
    out_type=jax.ShapeDtypeStruct((B_TOTAL, EMBED), jnp.float32),
    mesh=plsc.VectorSubcoreMesh(core_axis_name="c", subcore_axis_name="s"),
    scratch_types=[
        pltpu.VMEM((NCHUNKS, CHUNK), jnp.int32),
        pltpu.VMEM((NCHUNKS, CHUNK), jnp.int32),
        pltpu.VMEM((2, CHUNK, EMBED), jnp.float32),
        pltpu.VMEM((2, CHUNK, EMBED), jnp.float32),
        pltpu.VMEM((2, CHUNK, EMBED), jnp.float32),
        pltpu.SemaphoreType.DMA,
        pltpu.SemaphoreType.DMA,
        pltpu.SemaphoreType.DMA,
        pltpu.SemaphoreType.DMA,
        pltpu.SemaphoreType.DMA,
        pltpu.SemaphoreType.DMA,
    ],
    compiler_params=pltpu.CompilerParams(use_tc_tiling_on_sc=False),
)
def _sc_embed(seq_hbm, seg_hbm, tok_hbm, comb_hbm, out_hbm,
              sidx_v, cidx_v, tok_v, cmb_v, res_v,
              sem_t0, sem_t1, sem_c0, sem_c1, sem_w0, sem_w1):
    wid = lax.axis_index("s") * NC + lax.axis_index("c")
    gchunk0 = wid * NCHUNKS
    iota = lax.iota(jnp.int32, LANES)
    sem_t = (sem_t0, sem_t1)
    sem_c = (sem_c0, sem_c1)
    sem_w = (sem_w0, sem_w1)

    pltpu.sync_copy(seq_hbm.at[pl.ds(gchunk0, NCHUNKS)], sidx_v)
    pltpu.sync_copy(seg_hbm.at[pl.ds(gchunk0, NCHUNKS)], cidx_v)

    def cidx_body(c, _):
        for j in range(CHUNK // LANES):
            start = lax.rem(c * CHUNK + j * LANES, SEQ_LEN)
            traw = start + iota
            t = jnp.where(traw >= SEQ_LEN, traw - SEQ_LEN, traw)
            sl = pl.ds(j * LANES, LANES)
            cidx_v[c, sl] = cidx_v[c, sl] * SEQ_LEN + t
        return 0

    lax.fori_loop(0, NCHUNKS, cidx_body, 0)

    def gather(c, b):
        pltpu.async_copy(tok_hbm.at[sidx_v.at[c]], tok_v.at[b], sem_t[b])
        pltpu.async_copy(comb_hbm.at[cidx_v.at[c]], cmb_v.at[b], sem_c[b])

    gather(0, 0)
    gather(1, 1)

    def pair_body(g, _):
        for b in range(2):
            c = g * 2 + b
            pltpu.make_async_copy(tok_hbm.at[sidx_v.at[c]], tok_v.at[b], sem_t[b]).wait()
            pltpu.make_async_copy(comb_hbm.at[cidx_v.at[c]], cmb_v.at[b], sem_c[b]).wait()
            base = (gchunk0 + c) * CHUNK

            @pl.when(g >= 1)
            def _():
                prev = (gchunk0 + c - 2) * CHUNK
                pltpu.make_async_copy(
                    res_v.at[b], out_hbm.at[pl.ds(prev, CHUNK)], sem_w[b]
                ).wait()

            def add_body(r4, _):
                for rr in range(ROW_UNROLL):
                    for cc in range(CSL):
                        sl = pl.ds(cc * LANES, LANES)
                        res_v[b, r4 * ROW_UNROLL + rr, sl] = (
                            tok_v[b, r4 * ROW_UNROLL + rr, sl]
                            + cmb_v[b, r4 * ROW_UNROLL + rr, sl]
                        )
                return 0

            lax.fori_loop(0, CHUNK // ROW_UNROLL, add_body, 0)

            pltpu.async_copy(res_v.at[b], out_hbm.at[pl.ds(base, CHUNK)], sem_w[b])

            @pl.when(g < NCHUNKS // 2 - 1)
            def _():
                gather(c + 2, b)
        return 0

    lax.fori_loop(0, NCHUNKS // 2, pair_body, 0)

    for b in range(2):
        last = (gchunk0 + NCHUNKS - 2 + b) * CHUNK
        pltpu.make_async_copy(
            res_v.at[b], out_hbm.at[pl.ds(last, CHUNK)], sem_w[b]
        ).wait()


def kernel(sequence, segment_label, token_table, segment_table):
    nb, sl = sequence.shape
    pe = _make_pe(512, EMBED)[:sl]
    comb = (segment_table[:, None, :] + pe[None, :, :]).reshape(N_SEG * sl, EMBED)
    seq_flat = sequence.reshape(-1, CHUNK).astype(jnp.int32)
    seg_flat = segment_label.reshape(-1, CHUNK).astype(jnp.int32)
    out = _sc_embed(seq_flat, seg_flat, token_table, comb)
    return out.reshape(nb, sl, EMBED)
